# Initial kernel scaffold; baseline (speedup 1.0000x reference)
import jax
import jax.numpy as jnp
from jax.experimental import pallas as pl


def kernel(output, mask, ind, target):
    raise NotImplementedError("write your pallas kernel here")


# speedup vs baseline: 7.0460x; 7.0460x over previous
import jax
import jax.numpy as jnp
from jax import lax
from jax.experimental import pallas as pl
from jax.experimental.pallas import tpu as pltpu
from jax.experimental.pallas import tpu_sc as plsc

B, D, H, W = 16, 128, 128, 128
C = D // 2
HW = H * W
P = B * C
NW = 16
PPW = P // NW
L = 16


def _sc_body(out_hbm, ind_hbm, t_hbm, mask_hbm, part_hbm,
             ind_v, idx_v, vals_v, t_v, mask_v, acc_v, sem):
    cid = lax.axis_index("c")
    sid = lax.axis_index("s")

    @pl.when(cid == 0)
    def _work():
        base = sid * PPW
        pltpu.sync_copy(ind_hbm.at[pl.ds(base, PPW)], ind_v)
        pltpu.sync_copy(t_hbm.at[pl.ds(base, PPW)], t_v.at[pl.ds(0, PPW)])
        pltpu.sync_copy(t_hbm.at[pl.ds(P + base, PPW)], t_v.at[pl.ds(PPW, PPW)])
        pltpu.sync_copy(mask_hbm.at[pl.ds(base, PPW)], mask_v)

        iota = lax.iota(jnp.int32, L)
        for j in range(PPW // L):
            pair = base + j * L + iota
            f0 = pair * (2 * HW) + ind_v[pl.ds(j * L, L)]
            idx_v[pl.ds(j * L, L)] = f0
            idx_v[pl.ds(PPW + j * L, L)] = f0 + HW

        pltpu.async_copy(out_hbm.at[idx_v], vals_v, sem).wait()

        acc = jnp.zeros((L,), jnp.float32)
        for j in range(2 * PPW // L):
            v = vals_v[pl.ds(j * L, L)]
            t = t_v[pl.ds(j * L, L)]
            m = mask_v[pl.ds((j % (PPW // L)) * L, L)]
            acc = acc + jnp.abs(v * m - t * m)
        macc = jnp.zeros((L,), jnp.float32)
        for j in range(PPW // L):
            macc = macc + mask_v[pl.ds(j * L, L)]
        acc_v[pl.ds(0, L)] = acc
        acc_v[pl.ds(L, L)] = macc
        pltpu.sync_copy(acc_v, part_hbm.at[sid])


def _tc_reduce(part_ref, out_ref):
    p = part_ref[...]
    num = jnp.sum(p[:, :L])
    den = jnp.sum(p[:, L:])
    out_ref[...] = jnp.full((1, 1), num / (2.0 * den + 0.0001), jnp.float32)


@jax.jit
def _sc_loss(out_flat, ind_flat, t_cat, mask_flat):
    mesh = plsc.VectorSubcoreMesh(core_axis_name="c", subcore_axis_name="s")
    part = pl.kernel(
        _sc_body,
        out_type=jax.ShapeDtypeStruct((NW, 2 * L), jnp.float32),
        mesh=mesh,
        scratch_types=[
            pltpu.VMEM((PPW,), jnp.int32),
            pltpu.VMEM((2 * PPW,), jnp.int32),
            pltpu.VMEM((2 * PPW,), jnp.float32),
            pltpu.VMEM((2 * PPW,), jnp.float32),
            pltpu.VMEM((PPW,), jnp.float32),
            pltpu.VMEM((2 * L,), jnp.float32),
            pltpu.SemaphoreType.DMA,
        ],
    )(out_flat, ind_flat, t_cat, mask_flat)
    loss = pl.pallas_call(
        _tc_reduce,
        out_shape=jax.ShapeDtypeStruct((1, 1), jnp.float32),
    )(part)
    return loss[0, 0]


def kernel(output, mask, ind, target):
    assert output.shape == (B, D, H, W)
    out_flat = output.reshape(B * D * H * W)
    ind_flat = ind.reshape(P).astype(jnp.int32)
    mask_flat = mask.reshape(P).astype(jnp.float32)
    t = target.astype(jnp.float32)
    t_cat = jnp.concatenate([t[:, :, 0].reshape(P), t[:, :, 1].reshape(P)])
    return _sc_loss(out_flat, ind_flat, t_cat, mask_flat)
